# baseline (speedup 1.0000x reference)
import jax
import jax.numpy as jnp
from jax.experimental import pallas as pl
from jax.experimental.pallas import tpu as pltpu

_CHUNK = 1024
_IN_DEPTH = 8
_OUT_DEPTH = 4


def _row_start(g, B, S):
    p = g // B
    r = g % B
    return r * S + p * _CHUNK


def _make_body(B, S, D, NG, NP):
    def body(we_hbm, pos_hbm, o_hbm, we_buf, pos_buf, out_buf,
             we_sem, pos_sem, out_sem):
        def we_copy(g, slot):
            rs = _row_start(g, B, S)
            return pltpu.make_async_copy(
                we_hbm.at[pl.ds(rs, _CHUNK), :], we_buf.at[slot],
                we_sem.at[slot])

        def pos_copy(p, pslot):
            return pltpu.make_async_copy(
                pos_hbm.at[pl.ds(p * _CHUNK, _CHUNK), :], pos_buf.at[pslot],
                pos_sem.at[pslot])

        def out_copy(g, slot):
            rs = _row_start(g, B, S)
            return pltpu.make_async_copy(
                out_buf.at[slot], o_hbm.at[pl.ds(rs, _CHUNK), :],
                out_sem.at[slot])

        for k in range(_IN_DEPTH):
            we_copy(k, k).start()
        pos_copy(0, 0).start()
        pos_copy(1, 1).start()

        def step(g, _):
            islot = jax.lax.rem(g, _IN_DEPTH)
            oslot = jax.lax.rem(g, _OUT_DEPTH)
            p = g // B
            r = jax.lax.rem(g, B)
            pslot = jax.lax.rem(p, 2)

            we_copy(g, islot).wait()

            @pl.when(r == 0)
            def _():
                pos_copy(p, pslot).wait()

            @pl.when(g >= _OUT_DEPTH)
            def _():
                out_copy(g - _OUT_DEPTH, oslot).wait()

            out_buf[oslot] = we_buf[islot] + pos_buf[pslot]
            out_copy(g, oslot).start()

            @pl.when(g + _IN_DEPTH < NG)
            def _():
                we_copy(g + _IN_DEPTH, islot).start()

            @pl.when((r == B - 1) & (p + 2 < NP))
            def _():
                pos_copy(p + 2, pslot).start()

            return None

        jax.lax.fori_loop(0, NG, step, None)

        for k in range(_OUT_DEPTH):
            g_last = NG - _OUT_DEPTH + k
            out_copy(g_last, g_last % _OUT_DEPTH).wait()

    return body


def kernel(word_embeddings, pos_table):
    B, S, D = word_embeddings.shape
    we2 = word_embeddings.reshape(B * S, D)
    NG = (B * S) // _CHUNK
    NP = S // _CHUNK
    out = pl.pallas_call(
        _make_body(B, S, D, NG, NP),
        in_specs=[
            pl.BlockSpec(memory_space=pl.ANY),
            pl.BlockSpec(memory_space=pl.ANY),
        ],
        out_specs=pl.BlockSpec(memory_space=pl.ANY),
        out_shape=jax.ShapeDtypeStruct((B * S, D), word_embeddings.dtype),
        scratch_shapes=[
            pltpu.VMEM((_IN_DEPTH, _CHUNK, D), word_embeddings.dtype),
            pltpu.VMEM((2, _CHUNK, D), word_embeddings.dtype),
            pltpu.VMEM((_OUT_DEPTH, _CHUNK, D), word_embeddings.dtype),
            pltpu.SemaphoreType.DMA((_IN_DEPTH,)),
            pltpu.SemaphoreType.DMA((2,)),
            pltpu.SemaphoreType.DMA((_OUT_DEPTH,)),
        ],
    )(we2, pos_table)
    return out.reshape(B, S, D)


# speedup vs baseline: 1.0128x; 1.0128x over previous
import jax
import jax.numpy as jnp
from jax.experimental import pallas as pl
from jax.experimental.pallas import tpu as pltpu

_CHUNK = 1024
_IN_DEPTH = 4
_OUT_DEPTH = 8


def _row_start(g, B, S):
    p = g // B
    r = g % B
    return r * S + p * _CHUNK


def _make_body(B, S, D, NG, NP):
    def body(we_hbm, pos_hbm, o_hbm, we_buf, pos_buf, out_buf,
             we_sem, pos_sem, out_sem):
        def we_copy(g, slot):
            rs = _row_start(g, B, S)
            return pltpu.make_async_copy(
                we_hbm.at[pl.ds(rs, _CHUNK), :], we_buf.at[slot],
                we_sem.at[slot])

        def pos_copy(p, pslot):
            return pltpu.make_async_copy(
                pos_hbm.at[pl.ds(p * _CHUNK, _CHUNK), :], pos_buf.at[pslot],
                pos_sem.at[pslot])

        def out_copy(g, slot):
            rs = _row_start(g, B, S)
            return pltpu.make_async_copy(
                out_buf.at[slot], o_hbm.at[pl.ds(rs, _CHUNK), :],
                out_sem.at[slot])

        for k in range(_IN_DEPTH):
            we_copy(k, k).start()
        pos_copy(0, 0).start()
        pos_copy(1, 1).start()

        def step(g, _):
            islot = jax.lax.rem(g, _IN_DEPTH)
            oslot = jax.lax.rem(g, _OUT_DEPTH)
            p = g // B
            r = jax.lax.rem(g, B)
            pslot = jax.lax.rem(p, 2)

            we_copy(g, islot).wait()

            @pl.when(r == 0)
            def _():
                pos_copy(p, pslot).wait()

            @pl.when(g >= _OUT_DEPTH)
            def _():
                out_copy(g - _OUT_DEPTH, oslot).wait()

            out_buf[oslot] = we_buf[islot] + pos_buf[pslot]
            out_copy(g, oslot).start()

            @pl.when(g + _IN_DEPTH < NG)
            def _():
                we_copy(g + _IN_DEPTH, islot).start()

            @pl.when((r == B - 1) & (p + 2 < NP))
            def _():
                pos_copy(p + 2, pslot).start()

            return None

        jax.lax.fori_loop(0, NG, step, None)

        for k in range(_OUT_DEPTH):
            g_last = NG - _OUT_DEPTH + k
            out_copy(g_last, g_last % _OUT_DEPTH).wait()

    return body


def kernel(word_embeddings, pos_table):
    B, S, D = word_embeddings.shape
    we2 = word_embeddings.reshape(B * S, D)
    NG = (B * S) // _CHUNK
    NP = S // _CHUNK
    out = pl.pallas_call(
        _make_body(B, S, D, NG, NP),
        in_specs=[
            pl.BlockSpec(memory_space=pl.ANY),
            pl.BlockSpec(memory_space=pl.ANY),
        ],
        out_specs=pl.BlockSpec(memory_space=pl.ANY),
        out_shape=jax.ShapeDtypeStruct((B * S, D), word_embeddings.dtype),
        scratch_shapes=[
            pltpu.VMEM((_IN_DEPTH, _CHUNK, D), word_embeddings.dtype),
            pltpu.VMEM((2, _CHUNK, D), word_embeddings.dtype),
            pltpu.VMEM((_OUT_DEPTH, _CHUNK, D), word_embeddings.dtype),
            pltpu.SemaphoreType.DMA((_IN_DEPTH,)),
            pltpu.SemaphoreType.DMA((2,)),
            pltpu.SemaphoreType.DMA((_OUT_DEPTH,)),
        ],
    )(we2, pos_table)
    return out.reshape(B, S, D)


# speedup vs baseline: 1.0138x; 1.0009x over previous
import jax
import jax.numpy as jnp
from jax.experimental import pallas as pl
from jax.experimental.pallas import tpu as pltpu

_CHUNK = 1024
_IN_DEPTH = 5
_OUT_DEPTH = 7


def _row_start(g, B, S):
    p = g // B
    r = g % B
    return r * S + p * _CHUNK


def _make_body(B, S, D, NG, NP):
    def body(we_hbm, pos_hbm, o_hbm, we_buf, pos_buf, out_buf,
             we_sem, pos_sem, out_sem):
        def we_copy(g, slot):
            rs = _row_start(g, B, S)
            return pltpu.make_async_copy(
                we_hbm.at[pl.ds(rs, _CHUNK), :], we_buf.at[slot],
                we_sem.at[slot])

        def pos_copy(p, pslot):
            return pltpu.make_async_copy(
                pos_hbm.at[pl.ds(p * _CHUNK, _CHUNK), :], pos_buf.at[pslot],
                pos_sem.at[pslot])

        def out_copy(g, slot):
            rs = _row_start(g, B, S)
            return pltpu.make_async_copy(
                out_buf.at[slot], o_hbm.at[pl.ds(rs, _CHUNK), :],
                out_sem.at[slot])

        for k in range(_IN_DEPTH):
            we_copy(k, k).start()
        pos_copy(0, 0).start()
        pos_copy(1, 1).start()

        def step(g, _):
            islot = jax.lax.rem(g, _IN_DEPTH)
            oslot = jax.lax.rem(g, _OUT_DEPTH)
            p = g // B
            r = jax.lax.rem(g, B)
            pslot = jax.lax.rem(p, 2)

            we_copy(g, islot).wait()

            @pl.when(r == 0)
            def _():
                pos_copy(p, pslot).wait()

            @pl.when(g >= _OUT_DEPTH)
            def _():
                out_copy(g - _OUT_DEPTH, oslot).wait()

            out_buf[oslot] = we_buf[islot] + pos_buf[pslot]
            out_copy(g, oslot).start()

            @pl.when(g + _IN_DEPTH < NG)
            def _():
                we_copy(g + _IN_DEPTH, islot).start()

            @pl.when((r == B - 1) & (p + 2 < NP))
            def _():
                pos_copy(p + 2, pslot).start()

            return None

        jax.lax.fori_loop(0, NG, step, None)

        for k in range(_OUT_DEPTH):
            g_last = NG - _OUT_DEPTH + k
            out_copy(g_last, g_last % _OUT_DEPTH).wait()

    return body


def kernel(word_embeddings, pos_table):
    B, S, D = word_embeddings.shape
    we2 = word_embeddings.reshape(B * S, D)
    NG = (B * S) // _CHUNK
    NP = S // _CHUNK
    out = pl.pallas_call(
        _make_body(B, S, D, NG, NP),
        in_specs=[
            pl.BlockSpec(memory_space=pl.ANY),
            pl.BlockSpec(memory_space=pl.ANY),
        ],
        out_specs=pl.BlockSpec(memory_space=pl.ANY),
        out_shape=jax.ShapeDtypeStruct((B * S, D), word_embeddings.dtype),
        scratch_shapes=[
            pltpu.VMEM((_IN_DEPTH, _CHUNK, D), word_embeddings.dtype),
            pltpu.VMEM((2, _CHUNK, D), word_embeddings.dtype),
            pltpu.VMEM((_OUT_DEPTH, _CHUNK, D), word_embeddings.dtype),
            pltpu.SemaphoreType.DMA((_IN_DEPTH,)),
            pltpu.SemaphoreType.DMA((2,)),
            pltpu.SemaphoreType.DMA((_OUT_DEPTH,)),
        ],
    )(we2, pos_table)
    return out.reshape(B, S, D)


# speedup vs baseline: 1.0206x; 1.0067x over previous
import jax
import jax.numpy as jnp
from jax.experimental import pallas as pl
from jax.experimental.pallas import tpu as pltpu

_CHUNK = 1024
_IN_DEPTH = 6
_OUT_DEPTH = 6


def _row_start(g, B, S):
    p = g // B
    r = g % B
    return r * S + p * _CHUNK


def _make_body(B, S, D, NG, NP):
    def body(we_hbm, pos_hbm, o_hbm, we_buf, pos_buf, out_buf,
             we_sem, pos_sem, out_sem):
        def we_copy(g, slot):
            rs = _row_start(g, B, S)
            return pltpu.make_async_copy(
                we_hbm.at[pl.ds(rs, _CHUNK), :], we_buf.at[slot],
                we_sem.at[slot])

        def pos_copy(p, pslot):
            return pltpu.make_async_copy(
                pos_hbm.at[pl.ds(p * _CHUNK, _CHUNK), :], pos_buf.at[pslot],
                pos_sem.at[pslot])

        def out_copy(g, slot):
            rs = _row_start(g, B, S)
            return pltpu.make_async_copy(
                out_buf.at[slot], o_hbm.at[pl.ds(rs, _CHUNK), :],
                out_sem.at[slot])

        for k in range(_IN_DEPTH):
            we_copy(k, k).start()
        pos_copy(0, 0).start()
        pos_copy(1, 1).start()

        def step(g, _):
            islot = jax.lax.rem(g, _IN_DEPTH)
            oslot = jax.lax.rem(g, _OUT_DEPTH)
            p = g // B
            r = jax.lax.rem(g, B)
            pslot = jax.lax.rem(p, 2)

            we_copy(g, islot).wait()

            @pl.when(r == 0)
            def _():
                pos_copy(p, pslot).wait()

            @pl.when(g >= _OUT_DEPTH)
            def _():
                out_copy(g - _OUT_DEPTH, oslot).wait()

            out_buf[oslot] = we_buf[islot] + pos_buf[pslot]
            out_copy(g, oslot).start()

            @pl.when(g + _IN_DEPTH < NG)
            def _():
                we_copy(g + _IN_DEPTH, islot).start()

            @pl.when((r == B - 1) & (p + 2 < NP))
            def _():
                pos_copy(p + 2, pslot).start()

            return None

        jax.lax.fori_loop(0, NG, step, None)

        for k in range(_OUT_DEPTH):
            g_last = NG - _OUT_DEPTH + k
            out_copy(g_last, g_last % _OUT_DEPTH).wait()

    return body


def kernel(word_embeddings, pos_table):
    B, S, D = word_embeddings.shape
    we2 = word_embeddings.reshape(B * S, D)
    NG = (B * S) // _CHUNK
    NP = S // _CHUNK
    out = pl.pallas_call(
        _make_body(B, S, D, NG, NP),
        in_specs=[
            pl.BlockSpec(memory_space=pl.ANY),
            pl.BlockSpec(memory_space=pl.ANY),
        ],
        out_specs=pl.BlockSpec(memory_space=pl.ANY),
        out_shape=jax.ShapeDtypeStruct((B * S, D), word_embeddings.dtype),
        scratch_shapes=[
            pltpu.VMEM((_IN_DEPTH, _CHUNK, D), word_embeddings.dtype),
            pltpu.VMEM((2, _CHUNK, D), word_embeddings.dtype),
            pltpu.VMEM((_OUT_DEPTH, _CHUNK, D), word_embeddings.dtype),
            pltpu.SemaphoreType.DMA((_IN_DEPTH,)),
            pltpu.SemaphoreType.DMA((2,)),
            pltpu.SemaphoreType.DMA((_OUT_DEPTH,)),
        ],
    )(we2, pos_table)
    return out.reshape(B, S, D)


# speedup vs baseline: 1.0207x; 1.0002x over previous
import jax
from jax.experimental import pallas as pl
from jax.experimental.pallas import tpu as pltpu

_CHUNK = 1024
_IN_DEPTH = 6
_OUT_DEPTH = 6


def _row_start(g, B, S):
    p = g // B
    r = g % B
    return r * S + p * _CHUNK


def _make_body(B, S, D, NG, NP):
    def body(we_hbm, pos_hbm, o_hbm, we_buf, pos_buf, out_buf,
             we_sem, pos_sem, out_sem):
        def we_copy(g, slot):
            rs = _row_start(g, B, S)
            return pltpu.make_async_copy(
                we_hbm.at[pl.ds(rs, _CHUNK), :], we_buf.at[slot],
                we_sem.at[slot])

        def pos_copy(p, pslot):
            return pltpu.make_async_copy(
                pos_hbm.at[pl.ds(p * _CHUNK, _CHUNK), :], pos_buf.at[pslot],
                pos_sem.at[pslot])

        def out_copy(g, slot):
            rs = _row_start(g, B, S)
            return pltpu.make_async_copy(
                out_buf.at[slot], o_hbm.at[pl.ds(rs, _CHUNK), :],
                out_sem.at[slot])

        for k in range(_IN_DEPTH):
            we_copy(k, k).start()
        pos_copy(0, 0).start()
        pos_copy(1, 1).start()

        def step(g, _):
            islot = jax.lax.rem(g, _IN_DEPTH)
            oslot = jax.lax.rem(g, _OUT_DEPTH)
            p = g // B
            r = jax.lax.rem(g, B)
            pslot = jax.lax.rem(p, 2)

            we_copy(g, islot).wait()

            @pl.when(r == 0)
            def _():
                pos_copy(p, pslot).wait()

            @pl.when(g >= _OUT_DEPTH)
            def _():
                out_copy(g - _OUT_DEPTH, oslot).wait()

            out_buf[oslot] = we_buf[islot] + pos_buf[pslot]
            out_copy(g, oslot).start()

            @pl.when(g + _IN_DEPTH < NG)
            def _():
                we_copy(g + _IN_DEPTH, islot).start()

            @pl.when((r == B - 1) & (p + 2 < NP))
            def _():
                pos_copy(p + 2, pslot).start()

            return None

        jax.lax.fori_loop(0, NG, step, None)

        for k in range(_OUT_DEPTH):
            g_last = NG - _OUT_DEPTH + k
            out_copy(g_last, g_last % _OUT_DEPTH).wait()

    return body


def kernel(word_embeddings, pos_table):
    B, S, D = word_embeddings.shape
    we2 = word_embeddings.reshape(B * S, D)
    NG = (B * S) // _CHUNK
    NP = S // _CHUNK
    out = pl.pallas_call(
        _make_body(B, S, D, NG, NP),
        in_specs=[
            pl.BlockSpec(memory_space=pl.ANY),
            pl.BlockSpec(memory_space=pl.ANY),
        ],
        out_specs=pl.BlockSpec(memory_space=pl.ANY),
        out_shape=jax.ShapeDtypeStruct((B * S, D), word_embeddings.dtype),
        scratch_shapes=[
            pltpu.VMEM((_IN_DEPTH, _CHUNK, D), word_embeddings.dtype),
            pltpu.VMEM((2, _CHUNK, D), word_embeddings.dtype),
            pltpu.VMEM((_OUT_DEPTH, _CHUNK, D), word_embeddings.dtype),
            pltpu.SemaphoreType.DMA((_IN_DEPTH,)),
            pltpu.SemaphoreType.DMA((2,)),
            pltpu.SemaphoreType.DMA((_OUT_DEPTH,)),
        ],
    )(we2, pos_table)
    return out.reshape(B, S, D)
